# baseline (speedup 1.0000x reference)
import functools

import jax
import jax.numpy as jnp
from jax import lax
from jax.experimental import pallas as pl
from jax.experimental.pallas import tpu as pltpu
from jax.experimental.pallas import tpu_sc as plsc

NUM_CORES = 2
NUM_SUBCORES = 16
NUM_WORKERS = NUM_CORES * NUM_SUBCORES
B = 16384
D = 64
CHUNK = 128
CHUNKS_PER_WORKER = B // (NUM_WORKERS * CHUNK)
ROWS_PER_WORKER = B // NUM_WORKERS
LANES = 16
DGROUPS = D // LANES


def _sc_partials(idx, weight):
  mesh = plsc.VectorSubcoreMesh(
      core_axis_name="c", subcore_axis_name="s",
      num_cores=NUM_CORES, num_subcores=NUM_SUBCORES)

  @functools.partial(
      pl.kernel,
      out_type=jax.ShapeDtypeStruct((NUM_WORKERS, D), jnp.float32),
      mesh=mesh,
      compiler_params=pltpu.CompilerParams(use_tc_tiling_on_sc=False),
      scratch_types=[
          pltpu.VMEM((CHUNKS_PER_WORKER, CHUNK), jnp.int32),
          pltpu.VMEM((ROWS_PER_WORKER, D), jnp.float32),
          pltpu.VMEM((D,), jnp.float32),
          pltpu.SemaphoreType.DMA,
      ],
  )
  def body(idx_hbm, table_hbm, out_hbm, idx_v, rows_v, acc_v, sem):
    wid = lax.axis_index("s") * NUM_CORES + lax.axis_index("c")
    pltpu.sync_copy(idx_hbm.at[wid], idx_v)
    descs = []
    for j in range(CHUNKS_PER_WORKER):
      descs.append(pltpu.async_copy(
          table_hbm.at[idx_v.at[j]],
          rows_v.at[pl.ds(j * CHUNK, CHUNK)],
          sem))
    for d in descs:
      d.wait()

    def row_body(i, accs):
      return tuple(accs[g] + rows_v[i, pl.ds(g * LANES, LANES)]
                   for g in range(DGROUPS))

    init = tuple(jnp.zeros((LANES,), jnp.float32) for _ in range(DGROUPS))
    accs = lax.fori_loop(0, ROWS_PER_WORKER, row_body, init)
    for g in range(DGROUPS):
      acc_v[pl.ds(g * LANES, LANES)] = accs[g]
    pltpu.sync_copy(acc_v, out_hbm.at[wid])

  return body(idx, weight)


def _tc_combine(partials):
  def body(p_ref, o_ref):
    o_ref[...] = jnp.sum(p_ref[...], axis=0, keepdims=True) * (1.0 / B)

  return pl.pallas_call(
      body,
      out_shape=jax.ShapeDtypeStruct((1, D), jnp.float32),
  )(partials)


def kernel(input, weight):
  idx = input.reshape(NUM_WORKERS, CHUNKS_PER_WORKER, CHUNK).astype(jnp.int32)
  partials = _sc_partials(idx, weight)
  out = _tc_combine(partials)
  return out.reshape(1, 1, D)


# speedup vs baseline: 1.6905x; 1.6905x over previous
import functools

import jax
import jax.numpy as jnp
from jax import lax
from jax.experimental import pallas as pl
from jax.experimental.pallas import tpu as pltpu
from jax.experimental.pallas import tpu_sc as plsc

NUM_CORES = 2
NUM_SUBCORES = 16
NUM_WORKERS = NUM_CORES * NUM_SUBCORES
B = 16384
D = 64
ROWS_PER_WORKER = B // NUM_WORKERS
K = 32
NCHUNK = ROWS_PER_WORKER // K
LANES = 16
DGROUPS = D // LANES


def _sc_partials(idx, weight):
  mesh = plsc.VectorSubcoreMesh(
      core_axis_name="c", subcore_axis_name="s",
      num_cores=NUM_CORES, num_subcores=NUM_SUBCORES)

  @functools.partial(
      pl.kernel,
      out_type=jax.ShapeDtypeStruct((NUM_WORKERS, D), jnp.float32),
      mesh=mesh,
      compiler_params=pltpu.CompilerParams(use_tc_tiling_on_sc=True),
      scratch_types=[
          pltpu.VMEM((ROWS_PER_WORKER,), jnp.int32),
          pltpu.VMEM((2, K, D), jnp.float32),
          pltpu.VMEM((D,), jnp.float32),
          pltpu.SemaphoreType.DMA,
          pltpu.SemaphoreType.DMA,
      ],
  )
  def body(idx_hbm, table_hbm, out_hbm, idx_v, rows_v, acc_v, sem0, sem1):
    wid = lax.axis_index("s") * NUM_CORES + lax.axis_index("c")
    pltpu.sync_copy(idx_hbm.at[wid], idx_v)

    def fire(c, buf, sem):
      descs = []
      for kv in range(K // LANES):
        iv = idx_v[pl.ds(c * K + kv * LANES, LANES)]
        for l in range(LANES):
          r = iv[l]
          descs.append(pltpu.async_copy(
              table_hbm.at[r], rows_v.at[buf, kv * LANES + l], sem))
      return descs

    def accumulate(buf, accs):
      for k in range(K):
        accs = tuple(accs[g] + rows_v[buf, k, pl.ds(g * LANES, LANES)]
                     for g in range(DGROUPS))
      return accs

    def chunk_pair(i, accs):
      c0 = i * 2
      d0 = fire(c0, 0, sem0)
      d1 = fire(c0 + 1, 1, sem1)
      for d in d0:
        d.wait()
      accs = accumulate(0, accs)
      for d in d1:
        d.wait()
      accs = accumulate(1, accs)
      return accs

    init = tuple(jnp.zeros((LANES,), jnp.float32) for _ in range(DGROUPS))
    accs = lax.fori_loop(0, NCHUNK // 2, chunk_pair, init)
    for g in range(DGROUPS):
      acc_v[pl.ds(g * LANES, LANES)] = accs[g]
    pltpu.sync_copy(acc_v, out_hbm.at[wid])

  return body(idx, weight)


def _tc_combine(partials):
  def body(p_ref, o_ref):
    o_ref[...] = jnp.sum(p_ref[...], axis=0, keepdims=True) * (1.0 / B)

  return pl.pallas_call(
      body,
      out_shape=jax.ShapeDtypeStruct((1, D), jnp.float32),
  )(partials)


def kernel(input, weight):
  idx = input.reshape(NUM_WORKERS, ROWS_PER_WORKER).astype(jnp.int32)
  partials = _sc_partials(idx, weight)
  out = _tc_combine(partials)
  return out.reshape(1, 1, D)


# speedup vs baseline: 3.6886x; 2.1820x over previous
import functools

import jax
import jax.numpy as jnp
from jax import lax
from jax.experimental import pallas as pl
from jax.experimental.pallas import tpu as pltpu
from jax.experimental.pallas import tpu_sc as plsc

NUM_CORES = 2
NUM_SUBCORES = 16
NUM_WORKERS = NUM_CORES * NUM_SUBCORES
B = 16384
D = 64
VOCAB = 1000000
CPAD = 1 << 20
W16 = CPAD // NUM_SUBCORES
ZCH = 16384
CHUNK = 128
ROWS_PER_WORKER = B // NUM_WORKERS
NCH = ROWS_PER_WORKER // CHUNK
LANES = 16

BLK = 8192
NB = VOCAB // BLK
TAIL = VOCAB - NB * BLK


def _sc_counts(idx3):
  mesh = plsc.VectorSubcoreMesh(
      core_axis_name="c", subcore_axis_name="s",
      num_cores=NUM_CORES, num_subcores=NUM_SUBCORES)

  @functools.partial(
      pl.kernel,
      out_type=jax.ShapeDtypeStruct((NUM_CORES, CPAD), jnp.float32),
      mesh=mesh,
      compiler_params=pltpu.CompilerParams(use_tc_tiling_on_sc=True),
      scratch_types=[
          pltpu.VMEM((NCH, CHUNK), jnp.int32),
          pltpu.VMEM((ZCH,), jnp.float32),
          pltpu.VMEM((CHUNK,), jnp.float32),
          pltpu.VMEM_SHARED((CPAD,), jnp.float32),
      ],
  )
  def body(idx_hbm, out_hbm, idx_v, zero_v, ones_v, cnt_sh):
    cid = lax.axis_index("c")
    sid = lax.axis_index("s")
    wid = sid * NUM_CORES + cid

    pltpu.sync_copy(idx_hbm.at[wid], idx_v)

    def zstore(i, _):
      zero_v[pl.ds(i * LANES, LANES)] = jnp.zeros((LANES,), jnp.float32)
      return 0
    lax.fori_loop(0, ZCH // LANES, zstore, 0)
    for j in range(CHUNK // LANES):
      ones_v[pl.ds(j * LANES, LANES)] = jnp.ones((LANES,), jnp.float32)

    for j in range(W16 // ZCH):
      pltpu.sync_copy(zero_v, cnt_sh.at[pl.ds(sid * W16 + j * ZCH, ZCH)])
    plsc.subcore_barrier()

    for j in range(NCH):
      pltpu.sync_copy(ones_v, cnt_sh.at[idx_v.at[j]], add=True)
    plsc.subcore_barrier()

    pltpu.sync_copy(cnt_sh.at[pl.ds(sid * W16, W16)],
                    out_hbm.at[cid, pl.ds(sid * W16, W16)])

  return body(idx3)


GRID = (VOCAB + BLK - 1) // BLK


def _tc_scan(wt, counts):
  def body(wt_ref, c_ref, o_ref):
    q = pl.program_id(0)

    @pl.when(q == 0)
    def _():
      o_ref[...] = jnp.zeros((1, D), jnp.float32)

    cc = c_ref[0, :] + c_ref[1, :]
    cols = q * BLK + jax.lax.broadcasted_iota(jnp.int32, (1, BLK), 1)
    masked = jnp.where(cols < VOCAB, wt_ref[...] * cc[None, :], 0.0)
    o_ref[...] += jnp.sum(masked, axis=1).reshape(1, D)

    @pl.when(q == GRID - 1)
    def _():
      o_ref[...] *= 1.0 / B

  return pl.pallas_call(
      body,
      grid=(GRID,),
      in_specs=[pl.BlockSpec((D, BLK), lambda q: (0, q)),
                pl.BlockSpec((NUM_CORES, BLK), lambda q: (0, q))],
      out_specs=pl.BlockSpec((1, D), lambda q: (0, 0)),
      out_shape=jax.ShapeDtypeStruct((1, D), jnp.float32),
  )(wt, counts)


def kernel(input, weight):
  idx3 = input.reshape(NUM_WORKERS, NCH, CHUNK).astype(jnp.int32)
  counts = _sc_counts(idx3)
  out = _tc_scan(weight.T, counts)
  return out.reshape(1, 1, D)


# speedup vs baseline: 4.6465x; 1.2597x over previous
import functools

import jax
import jax.numpy as jnp
from jax import lax
from jax.experimental import pallas as pl
from jax.experimental.pallas import tpu as pltpu
from jax.experimental.pallas import tpu_sc as plsc

NUM_CORES = 2
NUM_SUBCORES = 16
NUM_WORKERS = NUM_CORES * NUM_SUBCORES
B = 16384
D = 64
VOCAB = 1000000
CPAD = 1 << 20
W16 = CPAD // NUM_SUBCORES
ZCH = 16384
CHUNK = 128
ROWS_PER_WORKER = B // NUM_WORKERS
NCH = ROWS_PER_WORKER // CHUNK
LANES = 16

BLK = 16384
NB = VOCAB // BLK
TAIL = VOCAB - NB * BLK


def _sc_counts(idx3):
  mesh = plsc.VectorSubcoreMesh(
      core_axis_name="c", subcore_axis_name="s",
      num_cores=NUM_CORES, num_subcores=NUM_SUBCORES)

  @functools.partial(
      pl.kernel,
      out_type=jax.ShapeDtypeStruct((NUM_CORES, CPAD), jnp.float32),
      mesh=mesh,
      compiler_params=pltpu.CompilerParams(use_tc_tiling_on_sc=True),
      scratch_types=[
          pltpu.VMEM((NCH, CHUNK), jnp.int32),
          pltpu.VMEM((ZCH,), jnp.float32),
          pltpu.VMEM((CHUNK,), jnp.float32),
          pltpu.VMEM_SHARED((CPAD,), jnp.float32),
      ],
  )
  def body(idx_hbm, out_hbm, idx_v, zero_v, ones_v, cnt_sh):
    cid = lax.axis_index("c")
    sid = lax.axis_index("s")
    wid = sid * NUM_CORES + cid

    pltpu.sync_copy(idx_hbm.at[wid], idx_v)

    def zstore(i, _):
      zero_v[pl.ds(i * LANES, LANES)] = jnp.zeros((LANES,), jnp.float32)
      return 0
    lax.fori_loop(0, ZCH // LANES, zstore, 0)
    for j in range(CHUNK // LANES):
      ones_v[pl.ds(j * LANES, LANES)] = jnp.ones((LANES,), jnp.float32)

    for j in range(W16 // ZCH):
      pltpu.sync_copy(zero_v, cnt_sh.at[pl.ds(sid * W16 + j * ZCH, ZCH)])
    plsc.subcore_barrier()

    for j in range(NCH):
      pltpu.sync_copy(ones_v, cnt_sh.at[idx_v.at[j]], add=True)
    plsc.subcore_barrier()

    pltpu.sync_copy(cnt_sh.at[pl.ds(sid * W16, W16)],
                    out_hbm.at[cid, pl.ds(sid * W16, W16)])

  return body(idx3)


GRID = (VOCAB + BLK - 1) // BLK


def _tc_scan(wt, counts):
  def body(wt_ref, c_ref, o_ref):
    q = pl.program_id(0)

    @pl.when(q == 0)
    def _():
      o_ref[...] = jnp.zeros((1, D), jnp.float32)

    cc = c_ref[0, :] + c_ref[1, :]
    cols = q * BLK + jax.lax.broadcasted_iota(jnp.int32, (1, BLK), 1)
    masked = jnp.where(cols < VOCAB, wt_ref[...] * cc[None, :], 0.0)
    o_ref[...] += jnp.sum(masked, axis=1).reshape(1, D)

    @pl.when(q == GRID - 1)
    def _():
      o_ref[...] *= 1.0 / B

  return pl.pallas_call(
      body,
      grid=(GRID,),
      in_specs=[pl.BlockSpec((D, BLK), lambda q: (0, q)),
                pl.BlockSpec((NUM_CORES, BLK), lambda q: (0, q))],
      out_specs=pl.BlockSpec((1, D), lambda q: (0, 0)),
      out_shape=jax.ShapeDtypeStruct((1, D), jnp.float32),
  )(wt, counts)


def kernel(input, weight):
  idx3 = input.reshape(NUM_WORKERS, NCH, CHUNK).astype(jnp.int32)
  counts = _sc_counts(idx3)
  out = _tc_scan(weight.T, counts)
  return out.reshape(1, 1, D)


# speedup vs baseline: 5.3171x; 1.1443x over previous
import functools

import jax
import jax.numpy as jnp
from jax import lax
from jax.experimental import pallas as pl
from jax.experimental.pallas import tpu as pltpu
from jax.experimental.pallas import tpu_sc as plsc

NUM_CORES = 2
NUM_SUBCORES = 16
NUM_WORKERS = NUM_CORES * NUM_SUBCORES
B = 16384
D = 64
VOCAB = 1000000
CPAD = 1 << 20
W16 = CPAD // NUM_SUBCORES
ZCH = 16384
CHUNK = 128
ROWS_PER_WORKER = B // NUM_WORKERS
NCH = ROWS_PER_WORKER // CHUNK
LANES = 16

BLK = 32768
NB = VOCAB // BLK
TAIL = VOCAB - NB * BLK


def _sc_counts(idx3):
  mesh = plsc.VectorSubcoreMesh(
      core_axis_name="c", subcore_axis_name="s",
      num_cores=NUM_CORES, num_subcores=NUM_SUBCORES)

  @functools.partial(
      pl.kernel,
      out_type=jax.ShapeDtypeStruct((NUM_CORES, CPAD), jnp.float32),
      mesh=mesh,
      compiler_params=pltpu.CompilerParams(use_tc_tiling_on_sc=True),
      scratch_types=[
          pltpu.VMEM((NCH, CHUNK), jnp.int32),
          pltpu.VMEM((ZCH,), jnp.float32),
          pltpu.VMEM((CHUNK,), jnp.float32),
          pltpu.VMEM_SHARED((CPAD,), jnp.float32),
      ],
  )
  def body(idx_hbm, out_hbm, idx_v, zero_v, ones_v, cnt_sh):
    cid = lax.axis_index("c")
    sid = lax.axis_index("s")
    wid = sid * NUM_CORES + cid

    pltpu.sync_copy(idx_hbm.at[wid], idx_v)

    def zstore(i, _):
      zero_v[pl.ds(i * LANES, LANES)] = jnp.zeros((LANES,), jnp.float32)
      return 0
    lax.fori_loop(0, ZCH // LANES, zstore, 0)
    for j in range(CHUNK // LANES):
      ones_v[pl.ds(j * LANES, LANES)] = jnp.ones((LANES,), jnp.float32)

    for j in range(W16 // ZCH):
      pltpu.sync_copy(zero_v, cnt_sh.at[pl.ds(sid * W16 + j * ZCH, ZCH)])
    plsc.subcore_barrier()

    for j in range(NCH):
      pltpu.sync_copy(ones_v, cnt_sh.at[idx_v.at[j]], add=True)
    plsc.subcore_barrier()

    pltpu.sync_copy(cnt_sh.at[pl.ds(sid * W16, W16)],
                    out_hbm.at[cid, pl.ds(sid * W16, W16)])

  return body(idx3)


GRID = (VOCAB + BLK - 1) // BLK


def _tc_scan(wt, counts):
  def body(wt_ref, c_ref, o_ref):
    q = pl.program_id(0)

    @pl.when(q == 0)
    def _():
      o_ref[...] = jnp.zeros((1, D), jnp.float32)

    cc = c_ref[0, :] + c_ref[1, :]
    cols = q * BLK + jax.lax.broadcasted_iota(jnp.int32, (1, BLK), 1)
    masked = jnp.where(cols < VOCAB, wt_ref[...] * cc[None, :], 0.0)
    o_ref[...] += jnp.sum(masked, axis=1).reshape(1, D)

    @pl.when(q == GRID - 1)
    def _():
      o_ref[...] *= 1.0 / B

  return pl.pallas_call(
      body,
      grid=(GRID,),
      in_specs=[pl.BlockSpec((D, BLK), lambda q: (0, q)),
                pl.BlockSpec((NUM_CORES, BLK), lambda q: (0, q))],
      out_specs=pl.BlockSpec((1, D), lambda q: (0, 0)),
      out_shape=jax.ShapeDtypeStruct((1, D), jnp.float32),
  )(wt, counts)


def kernel(input, weight):
  idx3 = input.reshape(NUM_WORKERS, NCH, CHUNK).astype(jnp.int32)
  counts = _sc_counts(idx3)
  out = _tc_scan(weight.T, counts)
  return out.reshape(1, 1, D)


# speedup vs baseline: 5.3822x; 1.0122x over previous
import functools

import jax
import jax.numpy as jnp
from jax import lax
from jax.experimental import pallas as pl
from jax.experimental.pallas import tpu as pltpu
from jax.experimental.pallas import tpu_sc as plsc

NUM_CORES = 2
NUM_SUBCORES = 16
NUM_WORKERS = NUM_CORES * NUM_SUBCORES
B = 16384
D = 64
VOCAB = 1000000
CPAD = 1 << 20
W16 = CPAD // NUM_SUBCORES
ZCH = 16384
CHUNK = 128
ROWS_PER_WORKER = B // NUM_WORKERS
NCH = ROWS_PER_WORKER // CHUNK
LANES = 16

BLK = 65536
NB = VOCAB // BLK
TAIL = VOCAB - NB * BLK


def _sc_counts(idx3):
  mesh = plsc.VectorSubcoreMesh(
      core_axis_name="c", subcore_axis_name="s",
      num_cores=NUM_CORES, num_subcores=NUM_SUBCORES)

  @functools.partial(
      pl.kernel,
      out_type=jax.ShapeDtypeStruct((NUM_CORES, CPAD), jnp.float32),
      mesh=mesh,
      compiler_params=pltpu.CompilerParams(use_tc_tiling_on_sc=True),
      scratch_types=[
          pltpu.VMEM((NCH, CHUNK), jnp.int32),
          pltpu.VMEM((ZCH,), jnp.float32),
          pltpu.VMEM((CHUNK,), jnp.float32),
          pltpu.VMEM_SHARED((CPAD,), jnp.float32),
      ],
  )
  def body(idx_hbm, out_hbm, idx_v, zero_v, ones_v, cnt_sh):
    cid = lax.axis_index("c")
    sid = lax.axis_index("s")
    wid = sid * NUM_CORES + cid

    pltpu.sync_copy(idx_hbm.at[wid], idx_v)

    def zstore(i, _):
      zero_v[pl.ds(i * LANES, LANES)] = jnp.zeros((LANES,), jnp.float32)
      return 0
    lax.fori_loop(0, ZCH // LANES, zstore, 0)
    for j in range(CHUNK // LANES):
      ones_v[pl.ds(j * LANES, LANES)] = jnp.ones((LANES,), jnp.float32)

    for j in range(W16 // ZCH):
      pltpu.sync_copy(zero_v, cnt_sh.at[pl.ds(sid * W16 + j * ZCH, ZCH)])
    plsc.subcore_barrier()

    for j in range(NCH):
      pltpu.sync_copy(ones_v, cnt_sh.at[idx_v.at[j]], add=True)
    plsc.subcore_barrier()

    pltpu.sync_copy(cnt_sh.at[pl.ds(sid * W16, W16)],
                    out_hbm.at[cid, pl.ds(sid * W16, W16)])

  return body(idx3)


GRID = (VOCAB + BLK - 1) // BLK


def _tc_scan(wt, counts):
  def body(wt_ref, c_ref, o_ref):
    q = pl.program_id(0)

    @pl.when(q == 0)
    def _():
      o_ref[...] = jnp.zeros((1, D), jnp.float32)

    cc = c_ref[0, :] + c_ref[1, :]
    cols = q * BLK + jax.lax.broadcasted_iota(jnp.int32, (1, BLK), 1)
    masked = jnp.where(cols < VOCAB, wt_ref[...] * cc[None, :], 0.0)
    o_ref[...] += jnp.sum(masked, axis=1).reshape(1, D)

    @pl.when(q == GRID - 1)
    def _():
      o_ref[...] *= 1.0 / B

  return pl.pallas_call(
      body,
      grid=(GRID,),
      in_specs=[pl.BlockSpec((D, BLK), lambda q: (0, q)),
                pl.BlockSpec((NUM_CORES, BLK), lambda q: (0, q))],
      out_specs=pl.BlockSpec((1, D), lambda q: (0, 0)),
      out_shape=jax.ShapeDtypeStruct((1, D), jnp.float32),
  )(wt, counts)


def kernel(input, weight):
  idx3 = input.reshape(NUM_WORKERS, NCH, CHUNK).astype(jnp.int32)
  counts = _sc_counts(idx3)
  out = _tc_scan(weight.T, counts)
  return out.reshape(1, 1, D)
